# baseline (speedup 1.0000x reference)
import functools

import jax
import jax.numpy as jnp
from jax import lax
from jax.experimental import pallas as pl
from jax.experimental.pallas import tpu as pltpu
from jax.experimental.pallas import tpu_sc as plsc

D_MODEL = 768
PERIOD = 86400

NW = 32
CHUNK = 32
N_PARTS = 4
BS = 512


def _sc_gather_body(b_per_w, n_chunks,
                    ts_hbm, table_hbm, out_hbm,
                    idx_v, rows0_v, rows1_v, rows2_v,
                    sg0, sg1, sg2, sw0, sw1, sw2):
    wid = lax.axis_index("s") * 2 + lax.axis_index("c")
    base = wid * b_per_w
    pltpu.sync_copy(ts_hbm.at[wid], idx_v)
    @pl.loop(0, b_per_w // 16)
    def _mod_loop(i):
        sl = pl.ds(i * 16, 16)
        t = idx_v[sl]
        r = lax.rem(t, PERIOD)
        idx_v[sl] = jnp.minimum(jnp.maximum(r, 0), PERIOD - 1)

    rows = (rows0_v, rows1_v, rows2_v)
    sem_g = (sg0, sg1, sg2)
    sem_w = (sw0, sw1, sw2)

    def start_gather(c):
        s = c % 3
        isl = idx_v.at[pl.ds(c * CHUNK, CHUNK)]
        return pltpu.async_copy(table_hbm.at[isl], rows[s], sem_g[s])

    def start_wb(c):
        s = c % 3
        dst = out_hbm.at[pl.ds(base + c * CHUNK, CHUNK)]
        return pltpu.async_copy(rows[s], dst, sem_w[s])

    gathers = [None] * n_chunks
    wbs = [None] * n_chunks
    gathers[0] = start_gather(0)
    if n_chunks > 1:
        gathers[1] = start_gather(1)
    for c in range(n_chunks):
        gathers[c].wait()
        wbs[c] = start_wb(c)
        if c + 2 < n_chunks:
            if c >= 1:
                wbs[c - 1].wait()
            gathers[c + 2] = start_gather(c + 2)
    if n_chunks >= 2:
        wbs[n_chunks - 2].wait()
    wbs[n_chunks - 1].wait()


def _sc_gather(ts, table):
    n_rows = ts.shape[0] * ts.shape[1]
    b_per_w = n_rows // NW
    k = pl.kernel(
        functools.partial(_sc_gather_body, b_per_w, b_per_w // CHUNK),
        out_type=jax.ShapeDtypeStruct((n_rows, D_MODEL), jnp.float32),
        mesh=plsc.VectorSubcoreMesh(core_axis_name="c", subcore_axis_name="s"),
        scratch_types=[
            pltpu.VMEM((b_per_w,), jnp.int32),
            pltpu.VMEM((CHUNK, D_MODEL), jnp.float32),
            pltpu.VMEM((CHUNK, D_MODEL), jnp.float32),
            pltpu.VMEM((CHUNK, D_MODEL), jnp.float32),
            pltpu.SemaphoreType.DMA,
            pltpu.SemaphoreType.DMA,
            pltpu.SemaphoreType.DMA,
            pltpu.SemaphoreType.DMA,
            pltpu.SemaphoreType.DMA,
            pltpu.SemaphoreType.DMA,
        ],
    )
    return k(ts, table)


def _tc_add_first_body(x_ref, pe_ref, c_ref, o_ref):
    o_ref[...] = x_ref[...] + pe_ref[...][None] + c_ref[...]


def _tc_add_acc_body(carry_ref, x_ref, pe_ref, c_ref, o_ref):
    o_ref[...] = x_ref[...] + pe_ref[...][None] + c_ref[...]


def _tc_add_part(p, x, pe, circ_p, carry):
    B, S, D = x.shape
    sp = S // N_PARTS
    nblk = sp // BS
    p0 = p * nblk
    x_spec = pl.BlockSpec((B, BS, D), lambda j: (0, p0 + j, 0))
    pe_spec = pl.BlockSpec((BS, D), lambda j: (p0 + j, 0))
    c_spec = pl.BlockSpec((B, BS, D), lambda j: (0, j, 0))
    o_spec = pl.BlockSpec((B, BS, D), lambda j: (0, p0 + j, 0))
    out_shape = jax.ShapeDtypeStruct((B, S, D), jnp.float32)
    if carry is None:
        return pl.pallas_call(
            _tc_add_first_body,
            grid=(nblk,),
            in_specs=[x_spec, pe_spec, c_spec],
            out_specs=o_spec,
            out_shape=out_shape,
        )(x, pe, circ_p)
    carry_spec = pl.BlockSpec(memory_space=pl.ANY)
    return pl.pallas_call(
        _tc_add_acc_body,
        grid=(nblk,),
        in_specs=[carry_spec, x_spec, pe_spec, c_spec],
        out_specs=o_spec,
        out_shape=out_shape,
        input_output_aliases={0: 0},
    )(carry, x, pe, circ_p)


def kernel(x, timestamps, pe, circadian_pe):
    B, S, D = x.shape
    assert D == D_MODEL and S % (N_PARTS * BS) == 0
    sp = S // N_PARTS
    ts = timestamps.astype(jnp.int32)
    circ = []
    for p in range(N_PARTS):
        ts_p = ts[:, p * sp:(p + 1) * sp].reshape(NW, (B * sp) // NW)
        c = _sc_gather(ts_p, circadian_pe)
        circ.append(c.reshape(B, sp, D))
    out = None
    for p in range(N_PARTS):
        out = _tc_add_part(p, x, pe, circ[p], out)
    return out


# speedup vs baseline: 1.0418x; 1.0418x over previous
import functools

import jax
import jax.numpy as jnp
from jax import lax
from jax.experimental import pallas as pl
from jax.experimental.pallas import tpu as pltpu
from jax.experimental.pallas import tpu_sc as plsc

D_MODEL = 768
PERIOD = 86400

NW = 32
CHUNK = 64
N_PARTS = 8
BS = 512


def _sc_gather_body(b_per_w, n_chunks,
                    ts_hbm, table_hbm, out_hbm,
                    idx_v, rows0_v, rows1_v, sg0, sg1, sw0, sw1):
    wid = lax.axis_index("s") * 2 + lax.axis_index("c")
    base = wid * b_per_w
    pltpu.sync_copy(ts_hbm.at[wid], idx_v)
    @pl.loop(0, b_per_w // 16)
    def _mod_loop(i):
        sl = pl.ds(i * 16, 16)
        t = idx_v[sl]
        r = lax.rem(t, PERIOD)
        idx_v[sl] = jnp.minimum(jnp.maximum(r, 0), PERIOD - 1)

    rows = (rows0_v, rows1_v)
    sem_g = (sg0, sg1)
    sem_w = (sw0, sw1)

    def start_gather(c, s):
        isl = idx_v.at[pl.ds(c * CHUNK, CHUNK)]
        return pltpu.async_copy(table_hbm.at[isl], rows[s], sem_g[s])

    def start_wb(c, s):
        dst = out_hbm.at[pl.ds(base + c * CHUNK, CHUNK)]
        return pltpu.async_copy(rows[s], dst, sem_w[s])

    gathers = [None] * n_chunks
    wbs = [None] * n_chunks
    gathers[0] = start_gather(0, 0)
    for c in range(n_chunks):
        s = c % 2
        if c + 1 < n_chunks:
            if c >= 1:
                wbs[c - 1].wait()
            gathers[c + 1] = start_gather(c + 1, s ^ 1)
        gathers[c].wait()
        wbs[c] = start_wb(c, s)
    if n_chunks >= 2:
        wbs[n_chunks - 2].wait()
    wbs[n_chunks - 1].wait()


def _sc_gather(ts, table):
    n_rows = ts.shape[0] * ts.shape[1]
    b_per_w = n_rows // NW
    k = pl.kernel(
        functools.partial(_sc_gather_body, b_per_w, b_per_w // CHUNK),
        out_type=jax.ShapeDtypeStruct((n_rows, D_MODEL), jnp.float32),
        mesh=plsc.VectorSubcoreMesh(core_axis_name="c", subcore_axis_name="s"),
        scratch_types=[
            pltpu.VMEM((b_per_w,), jnp.int32),
            pltpu.VMEM((CHUNK, D_MODEL), jnp.float32),
            pltpu.VMEM((CHUNK, D_MODEL), jnp.float32),
            pltpu.SemaphoreType.DMA,
            pltpu.SemaphoreType.DMA,
            pltpu.SemaphoreType.DMA,
            pltpu.SemaphoreType.DMA,
        ],
    )
    return k(ts, table)


def _tc_add_first_body(x_ref, pe_ref, c_ref, o_ref):
    o_ref[...] = x_ref[...] + pe_ref[...][None] + c_ref[...]


def _tc_add_acc_body(carry_ref, x_ref, pe_ref, c_ref, o_ref):
    o_ref[...] = x_ref[...] + pe_ref[...][None] + c_ref[...]


def _tc_add_part(p, x, pe, circ_p, carry):
    B, S, D = x.shape
    sp = S // N_PARTS
    nblk = sp // BS
    p0 = p * nblk
    x_spec = pl.BlockSpec((B, BS, D), lambda j: (0, p0 + j, 0))
    pe_spec = pl.BlockSpec((BS, D), lambda j: (p0 + j, 0))
    c_spec = pl.BlockSpec((B, BS, D), lambda j: (0, j, 0))
    o_spec = pl.BlockSpec((B, BS, D), lambda j: (0, p0 + j, 0))
    out_shape = jax.ShapeDtypeStruct((B, S, D), jnp.float32)
    if carry is None:
        return pl.pallas_call(
            _tc_add_first_body,
            grid=(nblk,),
            in_specs=[x_spec, pe_spec, c_spec],
            out_specs=o_spec,
            out_shape=out_shape,
        )(x, pe, circ_p)
    carry_spec = pl.BlockSpec(memory_space=pl.ANY)
    return pl.pallas_call(
        _tc_add_acc_body,
        grid=(nblk,),
        in_specs=[carry_spec, x_spec, pe_spec, c_spec],
        out_specs=o_spec,
        out_shape=out_shape,
        input_output_aliases={0: 0},
    )(carry, x, pe, circ_p)


def kernel(x, timestamps, pe, circadian_pe):
    B, S, D = x.shape
    assert D == D_MODEL and S % (N_PARTS * BS) == 0
    sp = S // N_PARTS
    ts = timestamps.astype(jnp.int32)
    circ = []
    for p in range(N_PARTS):
        ts_p = ts[:, p * sp:(p + 1) * sp].reshape(NW, (B * sp) // NW)
        c = _sc_gather(ts_p, circadian_pe)
        circ.append(c.reshape(B, sp, D))
    out = None
    for p in range(N_PARTS):
        out = _tc_add_part(p, x, pe, circ[p], out)
    return out
